# Initial kernel scaffold; baseline (speedup 1.0000x reference)
import jax
import jax.numpy as jnp
from jax.experimental import pallas as pl


def kernel(x, W):
    raise NotImplementedError("write your pallas kernel here")


# speedup vs baseline: 2.9789x; 2.9789x over previous
import functools

import jax
import jax.numpy as jnp
from jax.experimental import pallas as pl

_NUM_EMBEDDINGS = 1024
_EMBEDDING_DIM = 64
_BLOCK_ROWS = 2048


def _vq_block(x_ref, w_ref, q_ref, ste_ref):
    xb = x_ref[...]
    w = w_ref[...]
    scores = jax.lax.dot_general(
        xb, w, (((1,), (1,)), ((), ())),
        preferred_element_type=jnp.float32)
    xsq = jnp.sum(xb * xb, axis=1, keepdims=True)
    wsq = jnp.sum(w * w, axis=1)[None, :]
    dist = (xsq + wsq) - 2.0 * scores
    m = jnp.min(dist, axis=1, keepdims=True)
    col = jax.lax.broadcasted_iota(jnp.int32, dist.shape, 1)
    idx = jnp.min(jnp.where(dist == m, col, _NUM_EMBEDDINGS), axis=1)
    onehot = (col == idx[:, None]).astype(jnp.float32)
    q = jax.lax.dot_general(
        onehot, w, (((1,), (0,)), ((), ())),
        preferred_element_type=jnp.float32)
    q_ref[...] = q
    ste_ref[...] = (q - xb) + xb


@jax.jit
def kernel(x, W):
    n = x.shape[0] * x.shape[1]
    flat = x.reshape(n, _EMBEDDING_DIM)
    grid = (n // _BLOCK_ROWS,)
    q, ste = pl.pallas_call(
        _vq_block,
        grid=grid,
        in_specs=[
            pl.BlockSpec((_BLOCK_ROWS, _EMBEDDING_DIM), lambda i: (i, 0)),
            pl.BlockSpec((_NUM_EMBEDDINGS, _EMBEDDING_DIM), lambda i: (0, 0)),
        ],
        out_specs=[
            pl.BlockSpec((_BLOCK_ROWS, _EMBEDDING_DIM), lambda i: (i, 0)),
            pl.BlockSpec((_BLOCK_ROWS, _EMBEDDING_DIM), lambda i: (i, 0)),
        ],
        out_shape=[
            jax.ShapeDtypeStruct((n, _EMBEDDING_DIM), x.dtype),
            jax.ShapeDtypeStruct((n, _EMBEDDING_DIM), x.dtype),
        ],
    )(flat, W)
    return q.reshape(x.shape), ste.reshape(x.shape)


# speedup vs baseline: 3.1010x; 1.0410x over previous
import functools

import jax
import jax.numpy as jnp
from jax.experimental import pallas as pl

_NUM_EMBEDDINGS = 1024
_EMBEDDING_DIM = 64
_BLOCK_ROWS = 4096


def _vq_block(x_ref, w_ref, q_ref, ste_ref):
    xb = x_ref[...]
    w = w_ref[...]
    scores = jax.lax.dot_general(
        xb, w, (((1,), (1,)), ((), ())),
        preferred_element_type=jnp.float32)
    xsq = jnp.sum(xb * xb, axis=1, keepdims=True)
    wsq = jnp.sum(w * w, axis=1)[None, :]
    dist = (xsq + wsq) - 2.0 * scores
    idx = jnp.argmin(dist, axis=1).astype(jnp.int32)
    col = jax.lax.broadcasted_iota(jnp.int32, dist.shape, 1)
    onehot = (col == idx[:, None]).astype(jnp.float32)
    q = jax.lax.dot_general(
        onehot, w, (((1,), (0,)), ((), ())),
        preferred_element_type=jnp.float32)
    q_ref[...] = q
    ste_ref[...] = (q - xb) + xb


@jax.jit
def kernel(x, W):
    n = x.shape[0] * x.shape[1]
    flat = x.reshape(n, _EMBEDDING_DIM)
    grid = (n // _BLOCK_ROWS,)
    q, ste = pl.pallas_call(
        _vq_block,
        grid=grid,
        in_specs=[
            pl.BlockSpec((_BLOCK_ROWS, _EMBEDDING_DIM), lambda i: (i, 0)),
            pl.BlockSpec((_NUM_EMBEDDINGS, _EMBEDDING_DIM), lambda i: (0, 0)),
        ],
        out_specs=[
            pl.BlockSpec((_BLOCK_ROWS, _EMBEDDING_DIM), lambda i: (i, 0)),
            pl.BlockSpec((_BLOCK_ROWS, _EMBEDDING_DIM), lambda i: (i, 0)),
        ],
        out_shape=[
            jax.ShapeDtypeStruct((n, _EMBEDDING_DIM), x.dtype),
            jax.ShapeDtypeStruct((n, _EMBEDDING_DIM), x.dtype),
        ],
    )(flat, W)
    return q.reshape(x.shape), ste.reshape(x.shape)


# speedup vs baseline: 3.2408x; 1.0451x over previous
import functools

import jax
import jax.numpy as jnp
from jax.experimental import pallas as pl

_NUM_EMBEDDINGS = 1024
_EMBEDDING_DIM = 64
_BLOCK_ROWS = 4096


def _vq_block(x_ref, w_ref, q_ref, ste_ref):
    xb = x_ref[...]
    w = w_ref[...]
    scores = jax.lax.dot_general(
        xb, w, (((1,), (1,)), ((), ())),
        preferred_element_type=jnp.float32)
    xsq = jnp.sum(xb * xb, axis=1, keepdims=True)
    wsq = jnp.sum(w * w, axis=1)[None, :]
    dist = (xsq + wsq) - 2.0 * scores
    m = jnp.min(dist, axis=1, keepdims=True)
    col = jax.lax.broadcasted_iota(jnp.int32, dist.shape, 1).astype(jnp.float32)
    idx = jnp.min(jnp.where(dist == m, col, jnp.float32(_NUM_EMBEDDINGS)),
                  axis=1)
    onehot = (col == idx[:, None]).astype(jnp.float32)
    q = jax.lax.dot_general(
        onehot, w, (((1,), (0,)), ((), ())),
        preferred_element_type=jnp.float32)
    q_ref[...] = q
    ste_ref[...] = (q - xb) + xb


@jax.jit
def kernel(x, W):
    n = x.shape[0] * x.shape[1]
    flat = x.reshape(n, _EMBEDDING_DIM)
    grid = (n // _BLOCK_ROWS,)
    q, ste = pl.pallas_call(
        _vq_block,
        grid=grid,
        in_specs=[
            pl.BlockSpec((_BLOCK_ROWS, _EMBEDDING_DIM), lambda i: (i, 0)),
            pl.BlockSpec((_NUM_EMBEDDINGS, _EMBEDDING_DIM), lambda i: (0, 0)),
        ],
        out_specs=[
            pl.BlockSpec((_BLOCK_ROWS, _EMBEDDING_DIM), lambda i: (i, 0)),
            pl.BlockSpec((_BLOCK_ROWS, _EMBEDDING_DIM), lambda i: (i, 0)),
        ],
        out_shape=[
            jax.ShapeDtypeStruct((n, _EMBEDDING_DIM), x.dtype),
            jax.ShapeDtypeStruct((n, _EMBEDDING_DIM), x.dtype),
        ],
    )(flat, W)
    return q.reshape(x.shape), ste.reshape(x.shape)
